# baseline (speedup 1.0000x reference)
import functools

import jax
import jax.numpy as jnp
from jax.experimental import pallas as pl
from jax.experimental.pallas import tpu as pltpu

E = 64
TOPK = 2
H = 768
I = 768
S = 2048
N = S * TOPK
LIMIT = 7.0
ALPHA = 1.702

BT = 128
G = N // BT + E


def _moe_body(tile_e_ref, tot_ref, tokens_ref,
              w_ref, x_ref, bgu_ref, bdn_ref, wgu_hbm, wdn_hbm,
              out_ref,
              wgu_buf, wdn_buf, xs_ref, ys_ref, dsem):
    T = tot_ref[0]
    out_ref[...] = jnp.zeros_like(out_ref)

    def start_copy(i, slot):
        e = tile_e_ref[i]
        pltpu.make_async_copy(wgu_hbm.at[e], wgu_buf.at[slot],
                              dsem.at[slot, 0]).start()
        pltpu.make_async_copy(wdn_hbm.at[e], wdn_buf.at[slot],
                              dsem.at[slot, 1]).start()

    def wait_copy(i, slot):
        e = tile_e_ref[i]
        pltpu.make_async_copy(wgu_hbm.at[e], wgu_buf.at[slot],
                              dsem.at[slot, 0]).wait()
        pltpu.make_async_copy(wdn_hbm.at[e], wdn_buf.at[slot],
                              dsem.at[slot, 1]).wait()

    start_copy(0, 0)

    def tile_body(i, carry):
        slot = jax.lax.rem(i, 2)

        @pl.when(i + 1 < T)
        def _prefetch():
            start_copy(i + 1, 1 - slot)

        def gather_row(r, c):
            t = tokens_ref[i, r]
            xs_ref[r, :] = x_ref[t, :]
            return c
        jax.lax.fori_loop(0, BT, gather_row, 0, unroll=8)

        wait_copy(i, slot)
        e = tile_e_ref[i]
        xs = xs_ref[...]
        wgu = wgu_buf[slot]
        gu = jnp.dot(xs, wgu, preferred_element_type=jnp.float32)
        gu = gu + bgu_ref[e, :][None, :]
        gate = jnp.minimum(gu[:, :I], LIMIT)
        up = jnp.clip(gu[:, I:], -LIMIT, LIMIT)
        glu = gate * jax.nn.sigmoid(gate * ALPHA)
        h = (up + 1.0) * glu
        y = jnp.dot(h, wdn_buf[slot], preferred_element_type=jnp.float32)
        y = y + bdn_ref[e, :][None, :]
        ys_ref[...] = y * w_ref[i, 0, :][:, None]

        def scatter_row(r, c):
            t = tokens_ref[i, r]
            out_ref[pl.ds(t, 1), :] += ys_ref[pl.ds(r, 1), :]
            return c
        jax.lax.fori_loop(0, BT, scatter_row, 0, unroll=8)
        return carry

    jax.lax.fori_loop(0, T, tile_body, 0)


@functools.partial(jax.jit, static_argnames=())
def kernel(hidden_states, router_indices, routing_weights,
           W_gate_up, b_gate_up, W_down, b_down):
    x = hidden_states[0]
    experts = router_indices.reshape(N).astype(jnp.int32)
    w_flat = routing_weights.reshape(N)

    order = jnp.argsort(experts, stable=True)
    tok_sorted = (order // TOPK).astype(jnp.int32)
    w_sorted = w_flat[order]
    counts = jnp.bincount(experts, length=E).astype(jnp.int32)
    offsets = jnp.concatenate([jnp.zeros((1,), jnp.int32),
                               jnp.cumsum(counts)[:-1].astype(jnp.int32)])
    nt = (counts + BT - 1) // BT
    cum_nt = jnp.cumsum(nt).astype(jnp.int32)
    first_tile = cum_nt - nt
    total_tiles = cum_nt[-1:]
    gids = jnp.arange(G, dtype=jnp.int32)
    tile_e = jnp.searchsorted(cum_nt, gids, side='right').astype(jnp.int32)
    tile_e = jnp.minimum(tile_e, E - 1)
    tile_local = gids - first_tile[tile_e]
    tile_start = offsets[tile_e] + tile_local * BT
    tile_cnt = jnp.clip(counts[tile_e] - tile_local * BT, 0, BT).astype(jnp.int32)

    row_ids = tile_start[:, None] + jnp.arange(BT, dtype=jnp.int32)[None, :]
    row_valid = jnp.arange(BT, dtype=jnp.int32)[None, :] < tile_cnt[:, None]
    row_ids = jnp.clip(row_ids, 0, N - 1)
    tokens_tile = jnp.where(row_valid, tok_sorted[row_ids], 0)
    w_tile = jnp.where(row_valid, w_sorted[row_ids], 0.0)
    w_tile = w_tile.reshape(G, 1, BT)

    grid_spec = pltpu.PrefetchScalarGridSpec(
        num_scalar_prefetch=3,
        grid=(1,),
        in_specs=[
            pl.BlockSpec((G, 1, BT), lambda g, te, tot, tok: (0, 0, 0)),
            pl.BlockSpec((S, H), lambda g, te, tot, tok: (0, 0)),
            pl.BlockSpec((E, 2 * I), lambda g, te, tot, tok: (0, 0)),
            pl.BlockSpec((E, H), lambda g, te, tot, tok: (0, 0)),
            pl.BlockSpec(memory_space=pl.ANY),
            pl.BlockSpec(memory_space=pl.ANY),
        ],
        out_specs=pl.BlockSpec((S, H), lambda g, te, tot, tok: (0, 0)),
        scratch_shapes=[
            pltpu.VMEM((2, H, 2 * I), jnp.float32),
            pltpu.VMEM((2, I, H), jnp.float32),
            pltpu.VMEM((BT, H), jnp.float32),
            pltpu.VMEM((BT, H), jnp.float32),
            pltpu.SemaphoreType.DMA((2, 2)),
        ],
    )

    out = pl.pallas_call(
        _moe_body,
        grid_spec=grid_spec,
        out_shape=jax.ShapeDtypeStruct((S, H), jnp.float32),
        compiler_params=pltpu.CompilerParams(
            dimension_semantics=("arbitrary",),
        ),
    )(tile_e, total_tiles, tokens_tile,
      w_tile, x, b_gate_up, b_down, W_gate_up, W_down)

    return out.reshape(1, S, H)
